# baseline (speedup 1.0000x reference)
import functools

import jax
import jax.numpy as jnp
from jax import lax
from jax.experimental import pallas as pl
from jax.experimental.pallas import tpu as pltpu
from jax.experimental.pallas import tpu_sc as plsc

_THRESH = 0.02


def _conf_body(x0_ref, x1_ref, x2_ref, eh_ref, ridx_ref, p_ref):
    i = pl.program_id(0)
    BB = x0_ref.shape[1]
    Bn = BB * pl.num_programs(0)
    Hm1 = 3
    xs, exs = [], []
    for r in (x0_ref, x1_ref, x2_ref):
        x = r[0]
        m = jnp.max(x, axis=-1, keepdims=True)
        s = jnp.sum(jnp.exp(x - m), axis=-1)
        conf = 1.0 / s
        xs.append(x)
        exs.append(conf >= jnp.float32(_THRESH))
    eh = jnp.full((BB,), Hm1, jnp.int32)
    p = xs[Hm1 - 1]
    for h in range(Hm1 - 1, -1, -1):
        eh = jnp.where(exs[h], jnp.int32(h), eh)
        p = jnp.where((eh == h)[:, None], xs[h], p)
    bloc = jax.lax.broadcasted_iota(jnp.int32, (1, BB), 1)[0]
    eh_ref[...] = eh
    ridx_ref[...] = eh * Bn + i * BB + bloc
    p_ref[...] = p


def _exit_heads(logits):
    Hn, Bn, Cn = logits.shape
    BB = 1024
    specs = [
        pl.BlockSpec((1, BB, Cn), functools.partial(lambda h, i: (h, i, 0), h))
        for h in range(Hn - 1)
    ]
    return pl.pallas_call(
        _conf_body,
        grid=(Bn // BB,),
        in_specs=specs,
        out_specs=[
            pl.BlockSpec((BB,), lambda i: (i,)),
            pl.BlockSpec((BB,), lambda i: (i,)),
            pl.BlockSpec((BB, Cn), lambda i: (i, 0)),
        ],
        out_shape=[
            jax.ShapeDtypeStruct((Bn,), jnp.int32),
            jax.ShapeDtypeStruct((Bn,), jnp.int32),
            jax.ShapeDtypeStruct((Bn, Cn), logits.dtype),
        ],
    )(logits, logits, logits)


def _make_sc_route(Hn, Bn, Cn, dtype):
    info = plsc.get_sparse_core_info()
    NW = info.num_cores * info.num_subcores
    rows_per_w = Bn // NW
    G = 16
    n_g = rows_per_w // G
    last_base = (Hn - 1) * Bn
    mesh = plsc.VectorSubcoreMesh(core_axis_name="c", subcore_axis_name="s")

    @functools.partial(
        pl.kernel,
        mesh=mesh,
        out_type=jax.ShapeDtypeStruct((Bn, Cn), dtype),
        scratch_types=[
            pltpu.VMEM((rows_per_w,), jnp.int32),
            pltpu.SemaphoreType.DMA,
            pltpu.SemaphoreType.DMA,
        ],
    )
    def sc_route(table_hbm, p_hbm, ridx_hbm, out_hbm, idx_v, bsem, rsem):
        wid = lax.axis_index("s") * info.num_cores + lax.axis_index("c")
        base = wid * rows_per_w
        pltpu.sync_copy(ridx_hbm.at[pl.ds(base, rows_per_w)], idx_v)
        pltpu.async_copy(
            p_hbm.at[pl.ds(base, rows_per_w)],
            out_hbm.at[pl.ds(base, rows_per_w)],
            bsem,
        ).wait()

        def body(j, carry):
            vec = idx_v[pl.ds(j * G, G)]
            descs = []
            for r in range(G):
                row = vec[r]
                desc = pltpu.make_async_copy(
                    table_hbm.at[pl.ds(row, 1)],
                    out_hbm.at[pl.ds(base + j * G + r, 1)],
                    rsem,
                )

                @pl.when(row >= last_base)
                def _(desc=desc):
                    desc.start()

                descs.append((desc, row))
            for desc, row in descs:
                @pl.when(row >= last_base)
                def _(desc=desc):
                    desc.wait()
            return carry

        lax.fori_loop(0, n_g, body, 0)

    return sc_route


def kernel(logits):
    Hn, Bn, Cn = logits.shape
    eh, ridx, p = _exit_heads(logits)
    table = logits.reshape(Hn * Bn, Cn)
    out = _make_sc_route(Hn, Bn, Cn, logits.dtype)(table, p, ridx)
    return out, eh
